# Initial kernel scaffold; baseline (speedup 1.0000x reference)
import jax
import jax.numpy as jnp
from jax.experimental import pallas as pl


def kernel(x):
    raise NotImplementedError("write your pallas kernel here")


# speedup vs baseline: 2.2824x; 2.2824x over previous
import jax
import jax.numpy as jnp
from jax import lax
from jax.experimental import pallas as pl
from jax.experimental.pallas import tpu as pltpu

_T_LOW = 100.0
_T_HIGH = 200.0
_TAN22 = 0.41421356237309503
_TAN67 = 2.414213562373095
_TILE = 128


def _shx_zero(v, dx):
    if dx == 1:
        return jnp.concatenate([v[:, 1:], jnp.zeros_like(v[:, :1])], axis=1)
    if dx == -1:
        return jnp.concatenate([jnp.zeros_like(v[:, :1]), v[:, :-1]], axis=1)
    return v


def _shx_edge(v, dx):
    if dx == 1:
        return jnp.concatenate([v[:, 1:], v[:, -1:]], axis=1)
    if dx == -1:
        return jnp.concatenate([v[:, :1], v[:, :-1]], axis=1)
    return v


def _canny_kernel(x_ref, o_ref):
    H, W = x_ref.shape
    n_tiles = H // _TILE

    s0 = jnp.float32(0.0)
    for t in range(n_tiles):
        r0 = t * _TILE
        if t == 0:
            img = jnp.clip(jnp.floor(x_ref[0:_TILE + 2, :] * 255.0), 0.0, 255.0)
            img_ext = jnp.concatenate([img[:1], img[:1], img], axis=0)
        elif t == n_tiles - 1:
            img = jnp.clip(jnp.floor(x_ref[r0 - 2:H, :] * 255.0), 0.0, 255.0)
            img_ext = jnp.concatenate([img, img[-1:], img[-1:]], axis=0)
        else:
            img_ext = jnp.clip(jnp.floor(x_ref[r0 - 2:r0 + _TILE + 2, :] * 255.0),
                               0.0, 255.0)

        R = _TILE + 2
        sh = lambda dy, dx: _shx_edge(img_ext[1 + dy:1 + R + dy], dx)
        gx = (sh(-1, 1) + 2.0 * sh(0, 1) + sh(1, 1)) - \
             (sh(-1, -1) + 2.0 * sh(0, -1) + sh(1, -1))
        gy = (sh(1, -1) + 2.0 * sh(1, 0) + sh(1, 1)) - \
             (sh(-1, -1) + 2.0 * sh(-1, 0) + sh(-1, 1))
        ax = jnp.abs(gx)
        ay = jnp.abs(gy)
        mag = ax + ay

        c = slice(1, _TILE + 1)
        mag_c = mag[c]
        ax_c = ax[c]
        ay_c = ay[c]
        gxy_c = gx[c] * gy[c]

        nb = lambda dy, dx: _shx_zero(mag[1 + dy:_TILE + 1 + dy], dx)
        d0 = ay_c <= _TAN22 * ax_c
        d2 = ay_c > _TAN67 * ax_c
        d1 = (~d0) & (~d2) & (gxy_c > 0.0)
        n1 = jnp.where(d0, nb(0, 1),
                       jnp.where(d1, nb(-1, 1),
                                 jnp.where(d2, nb(-1, 0), nb(-1, -1))))
        n2 = jnp.where(d0, nb(0, -1),
                       jnp.where(d1, nb(1, -1),
                                 jnp.where(d2, nb(1, 0), nb(1, 1))))

        keep = (mag_c >= n1) & (mag_c > n2)
        ci = lax.broadcasted_iota(jnp.int32, (_TILE, W), 1)
        inter = (ci > 0) & (ci < W - 1)
        if t == 0:
            ri = lax.broadcasted_iota(jnp.int32, (_TILE, W), 0)
            inter = inter & (ri > 0)
        if t == n_tiles - 1:
            ri = lax.broadcasted_iota(jnp.int32, (_TILE, W), 0)
            inter = inter & (ri < _TILE - 1)
        keep = keep & inter

        state = jnp.where(keep,
                          jnp.where(mag_c > _T_HIGH, 2.0,
                                    jnp.where(mag_c > _T_LOW, 1.0, 0.0)),
                          0.0)
        o_ref[r0:r0 + _TILE, :] = state
        s0 = s0 + jnp.sum(state)

    def _update(t, want_sum):
        r0 = t * _TILE
        if t == 0:
            win = jnp.concatenate(
                [jnp.zeros((1, W), jnp.float32), o_ref[0:_TILE + 1, :]], axis=0)
        elif t == n_tiles - 1:
            win = jnp.concatenate(
                [o_ref[r0 - 1:H, :], jnp.zeros((1, W), jnp.float32)], axis=0)
        else:
            win = o_ref[r0 - 1:r0 + _TILE + 1, :]
        vm = jnp.maximum(jnp.maximum(win[0:_TILE], win[1:_TILE + 1]),
                         win[2:_TILE + 2])
        mx = jnp.maximum(jnp.maximum(vm, _shx_zero(vm, 1)), _shx_zero(vm, -1))
        cur = win[1:_TILE + 1]
        upd = jnp.where((cur == 1.0) & (mx > 1.5), 2.0, cur)
        o_ref[r0:r0 + _TILE, :] = upd
        if want_sum:
            return jnp.sum(upd)
        return jnp.float32(0.0)

    def _body(carry):
        _, prev = carry
        for t in range(n_tiles):
            _update(t, False)
        s = jnp.float32(0.0)
        for t in reversed(range(n_tiles)):
            s = s + _update(t, True)
        return (prev, s)

    def _cond(carry):
        a, b = carry
        return b > a

    lax.while_loop(_cond, _body, (jnp.float32(-1.0), s0))

    for t in range(n_tiles):
        r0 = t * _TILE
        v = o_ref[r0:r0 + _TILE, :]
        o_ref[r0:r0 + _TILE, :] = jnp.where(v > 1.5, 1.0, 0.0)


def _canny_pallas(x, interpret=False):
    H, W = x.shape
    return pl.pallas_call(
        _canny_kernel,
        out_shape=jax.ShapeDtypeStruct((H, W), jnp.float32),
        in_specs=[pl.BlockSpec(memory_space=pltpu.VMEM)],
        out_specs=pl.BlockSpec(memory_space=pltpu.VMEM),
        compiler_params=pltpu.CompilerParams(
            vmem_limit_bytes=56 * 1024 * 1024),
        name="canny_fused",
        interpret=interpret,
    )(x)


def kernel(x):
    H, W = x.shape
    edges = _canny_pallas(x)
    return jnp.broadcast_to(edges[None], (3, H, W))
